# baseline (speedup 1.0000x reference)
import functools

import jax
import jax.numpy as jnp
from jax import lax
from jax.experimental import pallas as pl
from jax.experimental.pallas import tpu as pltpu
from jax.experimental.pallas import tpu_sc as plsc

F32 = jnp.float32

_B, _C, _N, _K, _O = 8, 128, 2048, 20, 128
_BO = _B * _O
_NC = 2
_NS = 16
_NW = _NC * _NS
_PTS_W = _N // _NW
_PTS_CHUNK = 2
_NCHUNK = _PTS_W // _PTS_CHUNK
_ROWS = _PTS_CHUNK * _K
_LANES = 16
_CNT = float(_B * _N * _K)
_EPS = 1e-5

_HI = lax.Precision.HIGHEST


def _proj_body(x_ref, w_ref, g_ref, h_ref, hs_ref, hs2_ref):
    @pl.when(pl.program_id(0) == 0)
    def _init():
        hs_ref[...] = jnp.zeros_like(hs_ref)
        hs2_ref[...] = jnp.zeros_like(hs2_ref)

    w1 = w_ref[:, :_C]
    wd = w_ref[:, _C:] - w1
    dn = (((0,), (1,)), ((), ()))
    hs = jnp.zeros((1, _O), dtype=F32)
    hs2 = jnp.zeros((1, _O), dtype=F32)
    for b in range(_B):
        xb = x_ref[b]
        g_ref[:, b, :] = lax.dot_general(xb, w1, dn, preferred_element_type=F32,
                                         precision=_HI)
        h = lax.dot_general(xb, wd, dn, preferred_element_type=F32,
                            precision=_HI)
        h_ref[:, b, :] = h
        hs = hs + jnp.sum(h, axis=0, keepdims=True)
        hs2 = hs2 + jnp.sum(h * h, axis=0, keepdims=True)
    hs_ref[...] += float(_K) * hs
    hs2_ref[...] += float(_K) * hs2


def _project(x, W):
    nb = 512
    grid = (_N // nb,)
    return pl.pallas_call(
        _proj_body,
        grid=grid,
        in_specs=[
            pl.BlockSpec((_B, _C, nb), lambda n: (0, 0, n)),
            pl.BlockSpec((_O, 2 * _C), lambda n: (0, 0)),
        ],
        out_specs=[
            pl.BlockSpec((nb, _B, _O), lambda n: (n, 0, 0)),
            pl.BlockSpec((nb, _B, _O), lambda n: (n, 0, 0)),
            pl.BlockSpec((1, _O), lambda n: (0, 0)),
            pl.BlockSpec((1, _O), lambda n: (0, 0)),
        ],
        out_shape=[
            jax.ShapeDtypeStruct((_N, _B, _O), F32),
            jax.ShapeDtypeStruct((_N, _B, _O), F32),
            jax.ShapeDtypeStruct((1, _O), F32),
            jax.ShapeDtypeStruct((1, _O), F32),
        ],
    )(x, W)


def _tree(op, xs):
    while len(xs) > 1:
        nxt = [op(xs[i], xs[i + 1]) for i in range(0, len(xs) - 1, 2)]
        if len(xs) % 2:
            nxt.append(xs[-1])
        xs = nxt
    return xs[0]


def _sc_body(g_hbm, h_hbm, idx_hbm, m_hbm, sums_hbm,
             idx_all, rows_a, rows_b, h_a, h_b, m_a, m_b, acc_v, tmp_v, red_v,
             shared, sem_a, sem_b, sem_ha, sem_hb, sem_ma, sem_mb):
    cid = lax.axis_index("c")
    sid = lax.axis_index("s")
    wid = sid * _NC + cid
    ibase = wid * (_PTS_W * _K)
    nbase = wid * _PTS_W

    pltpu.sync_copy(idx_hbm.at[pl.ds(ibase, _PTS_W * _K)], idx_all)

    @pl.loop(0, 3 * _B)
    def _zrow(i):
        t = i // _B
        b = lax.rem(i, _B)

        @pl.loop(0, _O // _LANES)
        def _z(jo):
            acc_v[t, b, pl.ds(jo * _LANES, _LANES)] = jnp.zeros(
                (_LANES,), dtype=F32)

    def _gather(ci, rows_v, sem):
        return pltpu.make_async_copy(
            g_hbm.at[idx_all.at[pl.ds(ci * _ROWS, _ROWS)]], rows_v, sem)

    def _hfetch(ci, h_v, sem):
        return pltpu.make_async_copy(
            h_hbm.at[pl.ds(nbase + ci * _PTS_CHUNK, _PTS_CHUNK)], h_v, sem)

    def _compute(ci, rows_v, h_v, m_v, sem_m):
        @pl.loop(0, _B)
        def _bloop(b):
            @pl.loop(0, _O // _LANES, unroll=2)
            def _lane(jo):
                sl = pl.ds(jo * _LANES, _LANES)
                sa = acc_v[0, b, sl]
                qa = acc_v[1, b, sl]
                ca = acc_v[2, b, sl]
                for p in range(_PTS_CHUNK):
                    vs = [rows_v[p * _K + k, b, sl] for k in range(_K)]
                    s = _tree(lambda u, v: u + v, vs)
                    q = _tree(lambda u, v: u + v, [v * v for v in vs])
                    m_v[p, b, sl] = _tree(jnp.maximum, vs)
                    h = h_v[p, b, sl]
                    sa = sa + s
                    qa = qa + q
                    ca = ca + h * s
                acc_v[0, b, sl] = sa
                acc_v[1, b, sl] = qa
                acc_v[2, b, sl] = ca
        pltpu.async_copy(
            m_v, m_hbm.at[pl.ds(nbase + ci * _PTS_CHUNK, _PTS_CHUNK)], sem_m)

    def _mdrain(m_v, sem_m):
        pltpu.make_async_copy(m_v, m_hbm.at[pl.ds(0, _PTS_CHUNK)],
                              sem_m).wait()

    _gather(0, rows_a, sem_a).start()
    _hfetch(0, h_a, sem_ha).start()

    @pl.loop(0, _NCHUNK // 2)
    def _pair(ci2):
        c = ci2 * 2
        _gather(c + 1, rows_b, sem_b).start()
        _hfetch(c + 1, h_b, sem_hb).start()
        _gather(c, rows_a, sem_a).wait()
        _hfetch(c, h_a, sem_ha).wait()

        @pl.when(ci2 > 0)
        def _():
            _mdrain(m_a, sem_ma)
        _compute(c, rows_a, h_a, m_a, sem_ma)

        @pl.when(c + 2 < _NCHUNK)
        def _():
            _gather(c + 2, rows_a, sem_a).start()
            _hfetch(c + 2, h_a, sem_ha).start()

        _gather(c + 1, rows_b, sem_b).wait()
        _hfetch(c + 1, h_b, sem_hb).wait()

        @pl.when(ci2 > 0)
        def _():
            _mdrain(m_b, sem_mb)
        _compute(c + 1, rows_b, h_b, m_b, sem_mb)

    _mdrain(m_a, sem_ma)
    _mdrain(m_b, sem_mb)

    pltpu.sync_copy(acc_v, shared.at[sid])
    plsc.subcore_barrier()

    for r in range(2):
        pair = sid + _NS * r

        @pl.when(pair < 3 * _B)
        def _reduce():
            t = pair // _B
            b = lax.rem(pair, _B)
            pltpu.sync_copy(shared.at[:, t, b, :], tmp_v)

            @pl.loop(0, _O // _LANES)
            def _r(jo):
                sl = pl.ds(jo * _LANES, _LANES)
                red_v[sl] = _tree(lambda u, v: u + v,
                                  [tmp_v[g, sl] for g in range(_NS)])
            pltpu.sync_copy(red_v, sums_hbm.at[cid, t, b])


def _sc_gather_reduce(g3, h3, idx_flat):
    call = pl.kernel(
        _sc_body,
        out_type=(
            jax.ShapeDtypeStruct((_N, _B, _O), F32),
            jax.ShapeDtypeStruct((_NC, 3, _B, _O), F32),
        ),
        mesh=plsc.VectorSubcoreMesh(core_axis_name="c", subcore_axis_name="s"),
        scratch_types=[
            pltpu.VMEM((_PTS_W * _K,), jnp.int32),
            pltpu.VMEM((_ROWS, _B, _O), F32),
            pltpu.VMEM((_ROWS, _B, _O), F32),
            pltpu.VMEM((_PTS_CHUNK, _B, _O), F32),
            pltpu.VMEM((_PTS_CHUNK, _B, _O), F32),
            pltpu.VMEM((_PTS_CHUNK, _B, _O), F32),
            pltpu.VMEM((_PTS_CHUNK, _B, _O), F32),
            pltpu.VMEM((3, _B, _O), F32),
            pltpu.VMEM((_NS, _O), F32),
            pltpu.VMEM((_O,), F32),
            pltpu.VMEM_SHARED((_NS, 3, _B, _O), F32),
            pltpu.SemaphoreType.DMA,
            pltpu.SemaphoreType.DMA,
            pltpu.SemaphoreType.DMA,
            pltpu.SemaphoreType.DMA,
            pltpu.SemaphoreType.DMA,
            pltpu.SemaphoreType.DMA,
        ],
    )
    return call(g3, h3, idx_flat)


def _final_body(m_ref, h_ref, sums_ref, hs_ref, hs2_ref,
                gamma_ref, beta_ref, out_ref):
    part = sums_ref[0] + sums_ref[1]
    ssum = jnp.sum(part[0], axis=0, keepdims=True)
    qsum = jnp.sum(part[1], axis=0, keepdims=True)
    xsum = jnp.sum(part[2], axis=0, keepdims=True)
    tot = hs_ref[...] + ssum
    tsq = hs2_ref[...] + qsum + 2.0 * xsum
    mu = tot * (1.0 / _CNT)
    ex2 = tsq * (1.0 / _CNT)
    inv = lax.rsqrt(ex2 - mu * mu + _EPS)
    scale = gamma_ref[...] * inv
    shift = beta_ref[...] - mu * scale
    eye = jnp.eye(m_ref.shape[0], dtype=F32)
    for b in range(_B):
        y = m_ref[:, b, :] + h_ref[:, b, :]
        y = y * scale + shift
        y = jnp.where(y > 0, y, 0.2 * y)
        out_ref[b] = lax.dot_general(y, eye, (((0,), (0,)), ((), ())),
                                     preferred_element_type=F32, precision=_HI)


def _finalize(M3, H3, sums, hs, hs2, gamma2, beta2):
    nb = 128
    grid = (_N // nb,)
    return pl.pallas_call(
        _final_body,
        grid=grid,
        in_specs=[
            pl.BlockSpec((nb, _B, _O), lambda n: (n, 0, 0)),
            pl.BlockSpec((nb, _B, _O), lambda n: (n, 0, 0)),
            pl.BlockSpec((_NC, 3, _B, _O), lambda n: (0, 0, 0, 0)),
            pl.BlockSpec((1, _O), lambda n: (0, 0)),
            pl.BlockSpec((1, _O), lambda n: (0, 0)),
            pl.BlockSpec((1, _O), lambda n: (0, 0)),
            pl.BlockSpec((1, _O), lambda n: (0, 0)),
        ],
        out_specs=pl.BlockSpec((_B, _O, nb), lambda n: (0, 0, n)),
        out_shape=jax.ShapeDtypeStruct((_B, _O, _N), F32),
    )(M3, H3, sums, hs, hs2, gamma2, beta2)


def kernel(x, batch_indices, knn_idx, W, gamma, beta):
    del batch_indices
    G3, H3, hs, hs2 = _project(x, W)
    idx_flat = knn_idx.reshape(_N * _K)
    M3, sums = _sc_gather_reduce(G3, H3, idx_flat)
    gamma2 = gamma.reshape(1, _O)
    beta2 = beta.reshape(1, _O)
    return _finalize(M3, H3, sums, hs, hs2, gamma2, beta2)


# speedup vs baseline: 1.9442x; 1.9442x over previous
import functools

import jax
import jax.numpy as jnp
from jax import lax
from jax.experimental import pallas as pl
from jax.experimental.pallas import tpu as pltpu
from jax.experimental.pallas import tpu_sc as plsc

F32 = jnp.float32

_B, _C, _N, _K, _O = 8, 128, 2048, 20, 128
_BO = _B * _O
_NC = 2
_NS = 16
_NW = _NC * _NS
_PTS_W = _N // _NW
_PTS_CHUNK = 2
_NCHUNK = _PTS_W // _PTS_CHUNK
_ROWS = _PTS_CHUNK * _K
_LANES = 16
_CNT = float(_B * _N * _K)
_EPS = 1e-5

_HI = lax.Precision.HIGHEST


def _proj_body(x_ref, w_ref, g_ref, h_ref, hs_ref, hs2_ref):
    @pl.when(pl.program_id(0) == 0)
    def _init():
        hs_ref[...] = jnp.zeros_like(hs_ref)
        hs2_ref[...] = jnp.zeros_like(hs2_ref)

    w1 = w_ref[:, :_C]
    wd = w_ref[:, _C:] - w1
    dn = (((0,), (1,)), ((), ()))
    hs = jnp.zeros((1, _O), dtype=F32)
    hs2 = jnp.zeros((1, _O), dtype=F32)
    for b in range(_B):
        xb = x_ref[b]
        g_ref[:, b, :] = lax.dot_general(xb, w1, dn, preferred_element_type=F32,
                                         precision=_HI)
        h = lax.dot_general(xb, wd, dn, preferred_element_type=F32,
                            precision=_HI)
        h_ref[:, b, :] = h
        hs = hs + jnp.sum(h, axis=0, keepdims=True)
        hs2 = hs2 + jnp.sum(h * h, axis=0, keepdims=True)
    hs_ref[...] += float(_K) * hs
    hs2_ref[...] += float(_K) * hs2


def _project(x, W):
    nb = 512
    grid = (_N // nb,)
    return pl.pallas_call(
        _proj_body,
        grid=grid,
        in_specs=[
            pl.BlockSpec((_B, _C, nb), lambda n: (0, 0, n)),
            pl.BlockSpec((_O, 2 * _C), lambda n: (0, 0)),
        ],
        out_specs=[
            pl.BlockSpec((nb, _B, _O), lambda n: (n, 0, 0)),
            pl.BlockSpec((nb, _B, _O), lambda n: (n, 0, 0)),
            pl.BlockSpec((1, _O), lambda n: (0, 0)),
            pl.BlockSpec((1, _O), lambda n: (0, 0)),
        ],
        out_shape=[
            jax.ShapeDtypeStruct((_N, _B, _O), F32),
            jax.ShapeDtypeStruct((_N, _B, _O), F32),
            jax.ShapeDtypeStruct((1, _O), F32),
            jax.ShapeDtypeStruct((1, _O), F32),
        ],
    )(x, W)


def _tree(op, xs):
    while len(xs) > 1:
        nxt = [op(xs[i], xs[i + 1]) for i in range(0, len(xs) - 1, 2)]
        if len(xs) % 2:
            nxt.append(xs[-1])
        xs = nxt
    return xs[0]


def _sc_body(g_hbm, h_hbm, idx_hbm, m_hbm, sums_hbm,
             idx_all, rows_a, rows_b, h_a, h_b, m_a, m_b, acc_v, tmp_v, red_v,
             shared, sem_a, sem_b, sem_ha, sem_hb, sem_ma, sem_mb):
    cid = lax.axis_index("c")
    sid = lax.axis_index("s")
    wid = sid * _NC + cid
    ibase = wid * (_PTS_W * _K)
    nbase = wid * _PTS_W

    pltpu.sync_copy(idx_hbm.at[pl.ds(ibase, _PTS_W * _K)], idx_all)

    for t in range(3):
        for b in range(_B):
            @pl.loop(0, _O // _LANES)
            def _z(jo):
                acc_v[t, b, pl.ds(jo * _LANES, _LANES)] = jnp.zeros(
                    (_LANES,), dtype=F32)

    def _gather(ci, rows_v, sem):
        return pltpu.make_async_copy(
            g_hbm.at[idx_all.at[pl.ds(ci * _ROWS, _ROWS)]], rows_v, sem)

    def _hfetch(ci, h_v, sem):
        return pltpu.make_async_copy(
            h_hbm.at[pl.ds(nbase + ci * _PTS_CHUNK, _PTS_CHUNK)], h_v, sem)

    def _compute(ci, rows_v, h_v, m_v, sem_m):
        for b in range(_B):
            @pl.loop(0, _O // _LANES)
            def _lane(jo):
                sl = pl.ds(jo * _LANES, _LANES)
                sa = acc_v[0, b, sl]
                qa = acc_v[1, b, sl]
                ca = acc_v[2, b, sl]
                for p in range(_PTS_CHUNK):
                    vs = [rows_v[p * _K + k, b, sl] for k in range(_K)]
                    s = _tree(lambda u, v: u + v, vs)
                    q = _tree(lambda u, v: u + v, [v * v for v in vs])
                    m_v[p, b, sl] = _tree(jnp.maximum, vs)
                    h = h_v[p, b, sl]
                    sa = sa + s
                    qa = qa + q
                    ca = ca + h * s
                acc_v[0, b, sl] = sa
                acc_v[1, b, sl] = qa
                acc_v[2, b, sl] = ca
        pltpu.async_copy(
            m_v, m_hbm.at[pl.ds(nbase + ci * _PTS_CHUNK, _PTS_CHUNK)], sem_m)

    def _mdrain(m_v, sem_m):
        pltpu.make_async_copy(m_v, m_hbm.at[pl.ds(0, _PTS_CHUNK)],
                              sem_m).wait()

    _gather(0, rows_a, sem_a).start()
    _hfetch(0, h_a, sem_ha).start()

    @pl.loop(0, _NCHUNK // 2)
    def _pair(ci2):
        c = ci2 * 2
        _gather(c + 1, rows_b, sem_b).start()
        _hfetch(c + 1, h_b, sem_hb).start()
        _gather(c, rows_a, sem_a).wait()
        _hfetch(c, h_a, sem_ha).wait()

        @pl.when(ci2 > 0)
        def _():
            _mdrain(m_a, sem_ma)
        _compute(c, rows_a, h_a, m_a, sem_ma)

        @pl.when(c + 2 < _NCHUNK)
        def _():
            _gather(c + 2, rows_a, sem_a).start()
            _hfetch(c + 2, h_a, sem_ha).start()

        _gather(c + 1, rows_b, sem_b).wait()
        _hfetch(c + 1, h_b, sem_hb).wait()

        @pl.when(ci2 > 0)
        def _():
            _mdrain(m_b, sem_mb)
        _compute(c + 1, rows_b, h_b, m_b, sem_mb)

    _mdrain(m_a, sem_ma)
    _mdrain(m_b, sem_mb)

    pltpu.sync_copy(acc_v, shared.at[sid])
    plsc.subcore_barrier()

    for r in range(2):
        pair = sid + _NS * r

        @pl.when(pair < 3 * _B)
        def _reduce():
            t = pair // _B
            b = lax.rem(pair, _B)
            pltpu.sync_copy(shared.at[:, t, b, :], tmp_v)

            @pl.loop(0, _O // _LANES)
            def _r(jo):
                sl = pl.ds(jo * _LANES, _LANES)
                red_v[sl] = _tree(lambda u, v: u + v,
                                  [tmp_v[g, sl] for g in range(_NS)])
            pltpu.sync_copy(red_v, sums_hbm.at[cid, t, b])


def _sc_gather_reduce(g3, h3, idx_flat):
    call = pl.kernel(
        _sc_body,
        out_type=(
            jax.ShapeDtypeStruct((_N, _B, _O), F32),
            jax.ShapeDtypeStruct((_NC, 3, _B, _O), F32),
        ),
        mesh=plsc.VectorSubcoreMesh(core_axis_name="c", subcore_axis_name="s"),
        scratch_types=[
            pltpu.VMEM((_PTS_W * _K,), jnp.int32),
            pltpu.VMEM((_ROWS, _B, _O), F32),
            pltpu.VMEM((_ROWS, _B, _O), F32),
            pltpu.VMEM((_PTS_CHUNK, _B, _O), F32),
            pltpu.VMEM((_PTS_CHUNK, _B, _O), F32),
            pltpu.VMEM((_PTS_CHUNK, _B, _O), F32),
            pltpu.VMEM((_PTS_CHUNK, _B, _O), F32),
            pltpu.VMEM((3, _B, _O), F32),
            pltpu.VMEM((_NS, _O), F32),
            pltpu.VMEM((_O,), F32),
            pltpu.VMEM_SHARED((_NS, 3, _B, _O), F32),
            pltpu.SemaphoreType.DMA,
            pltpu.SemaphoreType.DMA,
            pltpu.SemaphoreType.DMA,
            pltpu.SemaphoreType.DMA,
            pltpu.SemaphoreType.DMA,
            pltpu.SemaphoreType.DMA,
        ],
    )
    return call(g3, h3, idx_flat)


def _final_body(m_ref, h_ref, sums_ref, hs_ref, hs2_ref,
                gamma_ref, beta_ref, out_ref):
    part = sums_ref[0] + sums_ref[1]
    ssum = jnp.sum(part[0], axis=0, keepdims=True)
    qsum = jnp.sum(part[1], axis=0, keepdims=True)
    xsum = jnp.sum(part[2], axis=0, keepdims=True)
    tot = hs_ref[...] + ssum
    tsq = hs2_ref[...] + qsum + 2.0 * xsum
    mu = tot * (1.0 / _CNT)
    ex2 = tsq * (1.0 / _CNT)
    inv = lax.rsqrt(ex2 - mu * mu + _EPS)
    scale = gamma_ref[...] * inv
    shift = beta_ref[...] - mu * scale
    eye = jnp.eye(m_ref.shape[0], dtype=F32)
    for b in range(_B):
        y = m_ref[:, b, :] + h_ref[:, b, :]
        y = y * scale + shift
        y = jnp.where(y > 0, y, 0.2 * y)
        out_ref[b] = lax.dot_general(y, eye, (((0,), (0,)), ((), ())),
                                     preferred_element_type=F32, precision=_HI)


def _finalize(M3, H3, sums, hs, hs2, gamma2, beta2):
    nb = 128
    grid = (_N // nb,)
    return pl.pallas_call(
        _final_body,
        grid=grid,
        in_specs=[
            pl.BlockSpec((nb, _B, _O), lambda n: (n, 0, 0)),
            pl.BlockSpec((nb, _B, _O), lambda n: (n, 0, 0)),
            pl.BlockSpec((_NC, 3, _B, _O), lambda n: (0, 0, 0, 0)),
            pl.BlockSpec((1, _O), lambda n: (0, 0)),
            pl.BlockSpec((1, _O), lambda n: (0, 0)),
            pl.BlockSpec((1, _O), lambda n: (0, 0)),
            pl.BlockSpec((1, _O), lambda n: (0, 0)),
        ],
        out_specs=pl.BlockSpec((_B, _O, nb), lambda n: (0, 0, n)),
        out_shape=jax.ShapeDtypeStruct((_B, _O, _N), F32),
    )(M3, H3, sums, hs, hs2, gamma2, beta2)


def kernel(x, batch_indices, knn_idx, W, gamma, beta):
    del batch_indices
    G3, H3, hs, hs2 = _project(x, W)
    idx_flat = knn_idx.reshape(_N * _K)
    M3, sums = _sc_gather_reduce(G3, H3, idx_flat)
    gamma2 = gamma.reshape(1, _O)
    beta2 = beta.reshape(1, _O)
    return _finalize(M3, H3, sums, hs, hs2, gamma2, beta2)


# speedup vs baseline: 2.0016x; 1.0295x over previous
import functools

import jax
import jax.numpy as jnp
from jax import lax
from jax.experimental import pallas as pl
from jax.experimental.pallas import tpu as pltpu
from jax.experimental.pallas import tpu_sc as plsc

F32 = jnp.float32

_B, _C, _N, _K, _O = 8, 128, 2048, 20, 128
_BO = _B * _O
_NC = 2
_NS = 16
_NW = _NC * _NS
_PTS_W = _N // _NW
_PTS_CHUNK = 2
_NCHUNK = _PTS_W // _PTS_CHUNK
_ROWS = _PTS_CHUNK * _K
_LANES = 16
_CNT = float(_B * _N * _K)
_EPS = 1e-5

_HI = lax.Precision.HIGHEST


def _proj_body(x_ref, w_ref, g_ref, h_ref, hs_ref, hs2_ref):
    @pl.when(pl.program_id(0) == 0)
    def _init():
        hs_ref[...] = jnp.zeros_like(hs_ref)
        hs2_ref[...] = jnp.zeros_like(hs2_ref)

    w1 = w_ref[:, :_C]
    wd = w_ref[:, _C:] - w1
    dn = (((0,), (1,)), ((), ()))
    hs = jnp.zeros((1, _O), dtype=F32)
    hs2 = jnp.zeros((1, _O), dtype=F32)
    for b in range(_B):
        xb = x_ref[b]
        g_ref[:, b, :] = lax.dot_general(xb, w1, dn, preferred_element_type=F32,
                                         precision=_HI)
        h = lax.dot_general(xb, wd, dn, preferred_element_type=F32,
                            precision=_HI)
        h_ref[:, b, :] = h
        hs = hs + jnp.sum(h, axis=0, keepdims=True)
        hs2 = hs2 + jnp.sum(h * h, axis=0, keepdims=True)
    hs_ref[...] += float(_K) * hs
    hs2_ref[...] += float(_K) * hs2


def _project(x, W):
    nb = 512
    grid = (_N // nb,)
    return pl.pallas_call(
        _proj_body,
        grid=grid,
        in_specs=[
            pl.BlockSpec((_B, _C, nb), lambda n: (0, 0, n)),
            pl.BlockSpec((_O, 2 * _C), lambda n: (0, 0)),
        ],
        out_specs=[
            pl.BlockSpec((nb, _B, _O), lambda n: (n, 0, 0)),
            pl.BlockSpec((nb, _B, _O), lambda n: (n, 0, 0)),
            pl.BlockSpec((1, _O), lambda n: (0, 0)),
            pl.BlockSpec((1, _O), lambda n: (0, 0)),
        ],
        out_shape=[
            jax.ShapeDtypeStruct((_N, _B, _O), F32),
            jax.ShapeDtypeStruct((_N, _B, _O), F32),
            jax.ShapeDtypeStruct((1, _O), F32),
            jax.ShapeDtypeStruct((1, _O), F32),
        ],
    )(x, W)


def _tree(op, xs):
    while len(xs) > 1:
        nxt = [op(xs[i], xs[i + 1]) for i in range(0, len(xs) - 1, 2)]
        if len(xs) % 2:
            nxt.append(xs[-1])
        xs = nxt
    return xs[0]


def _sc_body(g_hbm, h_hbm, idx_hbm, m_hbm, sums_hbm,
             idx_all, rows_a, rows_b, h_a, h_b, m_a, m_b, acc_v, tmp_v, red_v,
             shared, sem_a, sem_b, sem_ha, sem_hb, sem_ma, sem_mb):
    cid = lax.axis_index("c")
    sid = lax.axis_index("s")
    wid = sid * _NC + cid
    ibase = wid * (_PTS_W * _K)
    nbase = wid * _PTS_W

    pltpu.sync_copy(idx_hbm.at[pl.ds(ibase, _PTS_W * _K)], idx_all)

    for t in range(3):
        for b in range(_B):
            @pl.loop(0, _O // _LANES)
            def _z(jo):
                acc_v[t, b, pl.ds(jo * _LANES, _LANES)] = jnp.zeros(
                    (_LANES,), dtype=F32)

    def _gather(ci, rows_v, sem):
        return pltpu.make_async_copy(
            g_hbm.at[idx_all.at[pl.ds(ci * _ROWS, _ROWS)]], rows_v, sem)

    def _hfetch(ci, h_v, sem):
        return pltpu.make_async_copy(
            h_hbm.at[pl.ds(nbase + ci * _PTS_CHUNK, _PTS_CHUNK)], h_v, sem)

    def _compute(ci, rows_v, h_v, m_v, sem_m):
        for b in range(_B):
            @pl.loop(0, _O // _LANES)
            def _lane(jo):
                sl = pl.ds(jo * _LANES, _LANES)
                sa = acc_v[0, b, sl]
                qa = acc_v[1, b, sl]
                ca = acc_v[2, b, sl]
                for p in range(_PTS_CHUNK):
                    vs = [rows_v[p * _K + k, b, sl] for k in range(_K)]
                    s = _tree(lambda u, v: u + v, vs)
                    q = _tree(lambda u, v: u + v, [v * v for v in vs])
                    m_v[p, b, sl] = _tree(jnp.maximum, vs)
                    h = h_v[p, b, sl]
                    sa = sa + s
                    qa = qa + q
                    ca = ca + h * s
                acc_v[0, b, sl] = sa
                acc_v[1, b, sl] = qa
                acc_v[2, b, sl] = ca
        pltpu.async_copy(
            m_v, m_hbm.at[pl.ds(nbase + ci * _PTS_CHUNK, _PTS_CHUNK)], sem_m)

    def _mdrain(m_v, sem_m):
        pltpu.make_async_copy(m_v, m_hbm.at[pl.ds(0, _PTS_CHUNK)],
                              sem_m).wait()

    _gather(0, rows_a, sem_a).start()
    _hfetch(0, h_a, sem_ha).start()

    @pl.loop(0, _NCHUNK // 2)
    def _pair(ci2):
        c = ci2 * 2
        _gather(c + 1, rows_b, sem_b).start()
        _hfetch(c + 1, h_b, sem_hb).start()
        _gather(c, rows_a, sem_a).wait()
        _hfetch(c, h_a, sem_ha).wait()

        @pl.when(ci2 > 0)
        def _():
            _mdrain(m_a, sem_ma)
        _compute(c, rows_a, h_a, m_a, sem_ma)

        @pl.when(c + 2 < _NCHUNK)
        def _():
            _gather(c + 2, rows_a, sem_a).start()
            _hfetch(c + 2, h_a, sem_ha).start()

        _gather(c + 1, rows_b, sem_b).wait()
        _hfetch(c + 1, h_b, sem_hb).wait()

        @pl.when(ci2 > 0)
        def _():
            _mdrain(m_b, sem_mb)
        _compute(c + 1, rows_b, h_b, m_b, sem_mb)

    _mdrain(m_a, sem_ma)
    _mdrain(m_b, sem_mb)

    pltpu.sync_copy(acc_v, shared.at[sid])
    plsc.subcore_barrier()

    for r in range(2):
        pair = sid + _NS * r

        @pl.when(pair < 3 * _B)
        def _reduce():
            t = pair // _B
            b = lax.rem(pair, _B)
            pltpu.sync_copy(shared.at[:, t, b, :], tmp_v)

            @pl.loop(0, _O // _LANES)
            def _r(jo):
                sl = pl.ds(jo * _LANES, _LANES)
                red_v[sl] = _tree(lambda u, v: u + v,
                                  [tmp_v[g, sl] for g in range(_NS)])
            pltpu.sync_copy(red_v, sums_hbm.at[cid, t, b])


def _sc_gather_reduce(g3, h3, idx_flat):
    call = pl.kernel(
        _sc_body,
        out_type=(
            jax.ShapeDtypeStruct((_N, _B, _O), F32),
            jax.ShapeDtypeStruct((_NC, 3, _B, _O), F32),
        ),
        mesh=plsc.VectorSubcoreMesh(core_axis_name="c", subcore_axis_name="s"),
        scratch_types=[
            pltpu.VMEM((_PTS_W * _K,), jnp.int32),
            pltpu.VMEM((_ROWS, _B, _O), F32),
            pltpu.VMEM((_ROWS, _B, _O), F32),
            pltpu.VMEM((_PTS_CHUNK, _B, _O), F32),
            pltpu.VMEM((_PTS_CHUNK, _B, _O), F32),
            pltpu.VMEM((_PTS_CHUNK, _B, _O), F32),
            pltpu.VMEM((_PTS_CHUNK, _B, _O), F32),
            pltpu.VMEM((3, _B, _O), F32),
            pltpu.VMEM((_NS, _O), F32),
            pltpu.VMEM((_O,), F32),
            pltpu.VMEM_SHARED((_NS, 3, _B, _O), F32),
            pltpu.SemaphoreType.DMA,
            pltpu.SemaphoreType.DMA,
            pltpu.SemaphoreType.DMA,
            pltpu.SemaphoreType.DMA,
            pltpu.SemaphoreType.DMA,
            pltpu.SemaphoreType.DMA,
        ],
    )
    return call(g3, h3, idx_flat)


def _final_body(m_ref, h_ref, sums_ref, hs_ref, hs2_ref,
                gamma_ref, beta_ref, out_ref):
    part = sums_ref[0] + sums_ref[1]
    ssum = jnp.sum(part[0], axis=0, keepdims=True)
    qsum = jnp.sum(part[1], axis=0, keepdims=True)
    xsum = jnp.sum(part[2], axis=0, keepdims=True)
    tot = hs_ref[...] + ssum
    tsq = hs2_ref[...] + qsum + 2.0 * xsum
    mu = tot * (1.0 / _CNT)
    ex2 = tsq * (1.0 / _CNT)
    inv = lax.rsqrt(ex2 - mu * mu + _EPS)
    scale = gamma_ref[...] * inv
    shift = beta_ref[...] - mu * scale
    for b in range(_B):
        y = m_ref[:, b, :] + h_ref[:, b, :]
        y = y * scale + shift
        y = jnp.where(y > 0, y, 0.2 * y)
        out_ref[b] = y.T


def _finalize(M3, H3, sums, hs, hs2, gamma2, beta2):
    nb = 128
    grid = (_N // nb,)
    return pl.pallas_call(
        _final_body,
        grid=grid,
        in_specs=[
            pl.BlockSpec((nb, _B, _O), lambda n: (n, 0, 0)),
            pl.BlockSpec((nb, _B, _O), lambda n: (n, 0, 0)),
            pl.BlockSpec((_NC, 3, _B, _O), lambda n: (0, 0, 0, 0)),
            pl.BlockSpec((1, _O), lambda n: (0, 0)),
            pl.BlockSpec((1, _O), lambda n: (0, 0)),
            pl.BlockSpec((1, _O), lambda n: (0, 0)),
            pl.BlockSpec((1, _O), lambda n: (0, 0)),
        ],
        out_specs=pl.BlockSpec((_B, _O, nb), lambda n: (0, 0, n)),
        out_shape=jax.ShapeDtypeStruct((_B, _O, _N), F32),
    )(M3, H3, sums, hs, hs2, gamma2, beta2)


def kernel(x, batch_indices, knn_idx, W, gamma, beta):
    del batch_indices
    G3, H3, hs, hs2 = _project(x, W)
    idx_flat = knn_idx.reshape(_N * _K)
    M3, sums = _sc_gather_reduce(G3, H3, idx_flat)
    gamma2 = gamma.reshape(1, _O)
    beta2 = beta.reshape(1, _O)
    return _finalize(M3, H3, sums, hs, hs2, gamma2, beta2)
